# Initial kernel scaffold; baseline (speedup 1.0000x reference)
import jax
import jax.numpy as jnp
from jax.experimental import pallas as pl


def kernel(x, edge_index, edge_attr, conditions, scale, batch, params):
    raise NotImplementedError("write your pallas kernel here")


# speedup vs baseline: 2.9819x; 2.9819x over previous
import jax
import jax.numpy as jnp
from jax import lax
from jax.experimental import pallas as pl
from jax.experimental.pallas import tpu as pltpu
from jax.experimental.pallas import tpu_sc as plsc

N = 50000
E = 800000
B = 16
H = 64

NC = 2
NS = 16
NW = NC * NS

CHUNK = 128
NCHUNKS = E // CHUNK
RPT = 3128
NPAD = NS * RPT

BN = 5000
BE = 4000


def _sc_mesh():
    return plsc.VectorSubcoreMesh(core_axis_name="c", subcore_axis_name="s")


_SC_PARAMS = pltpu.CompilerParams(use_tc_tiling_on_sc=False)


def _gather_body(a_hbm, b_hbm, row_hbm, col_hbm, out_hbm,
                 idxr_v, idxc_v, ar_v, bc_v, sem1, sem2):
    wid = lax.axis_index("s") * NC + lax.axis_index("c")
    nk = (NCHUNKS + NW - 1) // NW

    def step(k, _):
        j = wid + k * NW

        @pl.when(j < NCHUNKS)
        def _():
            off = j * CHUNK
            pltpu.sync_copy(row_hbm.at[pl.ds(off, CHUNK)], idxr_v)
            pltpu.sync_copy(col_hbm.at[pl.ds(off, CHUNK)], idxc_v)
            cp1 = pltpu.async_copy(a_hbm.at[idxr_v], ar_v, sem1)
            cp2 = pltpu.async_copy(b_hbm.at[idxc_v], bc_v, sem2)
            cp1.wait()
            cp2.wait()

            def addrow(i, _):
                for q in range(H // 16):
                    sl = (i, pl.ds(q * 16, 16))
                    ar_v[sl] = ar_v[sl] + bc_v[sl]
                return 0

            lax.fori_loop(0, CHUNK, addrow, 0)
            pltpu.sync_copy(ar_v, out_hbm.at[pl.ds(off, CHUNK)])

        return 0

    lax.fori_loop(0, nk, step, 0)


def _sc_gather(a, b, row, col):
    k = pl.kernel(
        _gather_body,
        mesh=_sc_mesh(),
        compiler_params=_SC_PARAMS,
        out_type=jax.ShapeDtypeStruct((E, H), jnp.float32),
        scratch_types=[
            pltpu.VMEM((CHUNK,), jnp.int32),
            pltpu.VMEM((CHUNK,), jnp.int32),
            pltpu.VMEM((CHUNK, H), jnp.float32),
            pltpu.VMEM((CHUNK, H), jnp.float32),
            pltpu.SemaphoreType.DMA,
            pltpu.SemaphoreType.DMA,
        ],
    )
    return k(a, b, row, col)


def _scatter_body(e_hbm, row_hbm, out_hbm, acc_sh, stage_v, idx_v, vals_v):
    c = lax.axis_index("c")
    s = lax.axis_index("s")
    nk = (NCHUNKS + NS - 1) // NS

    for q in range(2):
        qi = c * 2 + q

        def zrow(i, _):
            stage_v[(i, pl.ds(0, 16))] = jnp.zeros((16,), jnp.float32)
            return 0

        lax.fori_loop(0, RPT, zrow, 0)
        pltpu.sync_copy(stage_v, acc_sh.at[pl.ds(s * RPT, RPT)])
        plsc.subcore_barrier()

        def step(k, _):
            j = s + k * NS

            @pl.when(j < NCHUNKS)
            def _():
                off = j * CHUNK
                pltpu.sync_copy(row_hbm.at[pl.ds(off, CHUNK)], idx_v)
                pltpu.sync_copy(
                    e_hbm.at[pl.ds(off, CHUNK), pl.ds(qi * 16, 16)], vals_v)
                pltpu.sync_copy(vals_v, acc_sh.at[idx_v], add=True)

            return 0

        lax.fori_loop(0, nk, step, 0)
        plsc.subcore_barrier()
        pltpu.sync_copy(acc_sh.at[pl.ds(s * RPT, RPT)], stage_v)
        pltpu.sync_copy(stage_v,
                        out_hbm.at[pl.ds(s * RPT, RPT), pl.ds(qi * 16, 16)])
        plsc.subcore_barrier()


def _sc_scatter(e, row):
    k = pl.kernel(
        _scatter_body,
        mesh=_sc_mesh(),
        compiler_params=_SC_PARAMS,
        out_type=jax.ShapeDtypeStruct((NPAD, H), jnp.float32),
        scratch_types=[
            pltpu.VMEM_SHARED((NPAD, 16), jnp.float32),
            pltpu.VMEM((RPT, 16), jnp.float32),
            pltpu.VMEM((CHUNK,), jnp.int32),
            pltpu.VMEM((CHUNK, 16), jnp.float32),
        ],
    )
    return k(e, row)[:N]


def _deg_body(row_hbm, out_hbm, acc_sh, stage_v, idx_v, ones_v):
    c = lax.axis_index("c")
    s = lax.axis_index("s")
    wid = s * NC + c

    def zrow(i, _):
        stage_v[(i, pl.ds(0, 16))] = jnp.zeros((16,), jnp.float32)
        return 0

    lax.fori_loop(0, RPT, zrow, 0)
    pltpu.sync_copy(stage_v, acc_sh.at[pl.ds(s * RPT, RPT)])

    def orow(i, _):
        ones_v[(i, pl.ds(0, 16))] = jnp.ones((16,), jnp.float32)
        return 0

    lax.fori_loop(0, CHUNK, orow, 0)
    plsc.subcore_barrier()

    nk = (NCHUNKS + NW - 1) // NW

    def step(k, _):
        j = wid + k * NW

        @pl.when(j < NCHUNKS)
        def _():
            off = j * CHUNK
            pltpu.sync_copy(row_hbm.at[pl.ds(off, CHUNK)], idx_v)
            pltpu.sync_copy(ones_v, acc_sh.at[idx_v], add=True)

        return 0

    lax.fori_loop(0, nk, step, 0)
    plsc.subcore_barrier()
    pltpu.sync_copy(acc_sh.at[pl.ds(s * RPT, RPT)], stage_v)
    pltpu.sync_copy(stage_v,
                    out_hbm.at[pl.ds(s * RPT, RPT), pl.ds(c * 16, 16)])


def _sc_degree(row):
    k = pl.kernel(
        _deg_body,
        mesh=_sc_mesh(),
        compiler_params=_SC_PARAMS,
        out_type=jax.ShapeDtypeStruct((NPAD, 32), jnp.float32),
        scratch_types=[
            pltpu.VMEM_SHARED((NPAD, 16), jnp.float32),
            pltpu.VMEM((RPT, 16), jnp.float32),
            pltpu.VMEM((CHUNK,), jnp.int32),
            pltpu.VMEM((CHUNK, 16), jnp.float32),
        ],
    )
    return k(row)[:N]


def _full(shape):
    return pl.BlockSpec(shape, lambda *_: tuple(0 for _ in shape))


def _dotT(a, b):
    return lax.dot_general(a, b, (((0,), (0,)), ((), ())),
                           preferred_element_type=jnp.float32)


def _prep_u_body(cond_ref, scl_ref, cw1, cb1, cw2, cb2, sw1, sb1, sw2, sb2,
                 uw1, ub1, uw2, ub2, wd_ref, bde_ref, wnc_ref, bnc_ref,
                 u_ref, te_ref, tn_ref):
    uc = jnp.maximum(cond_ref[...] @ cw1[...] + cb1[...], 0.0) @ cw2[...] + cb2[...]
    us = jnp.maximum(scl_ref[...] @ sw1[...] + sb1[...], 0.0) @ sw2[...] + sb2[...]
    cat = jnp.concatenate([uc, us], axis=1)
    u = jnp.maximum(cat @ uw1[...] + ub1[...], 0.0) @ uw2[...] + ub2[...]
    u_ref[...] = u
    for l in range(3):
        te_ref[l] = u @ wd_ref[l] + bde_ref[l]
        tn_ref[l] = u @ wnc_ref[l] + bnc_ref[l]


def _tc_prep_u(conditions, scale, p, wd, bde, wnc, bnc):
    outs = (
        jax.ShapeDtypeStruct((B, H), jnp.float32),
        jax.ShapeDtypeStruct((3, B, H), jnp.float32),
        jax.ShapeDtypeStruct((3, B, H), jnp.float32),
    )
    ce, se, ue = p["cond_enc"], p["scale_enc"], p["u_enc"]
    args = [conditions, scale,
            ce["l1"]["W"], ce["l1"]["b"][None, :], ce["l2"]["W"], ce["l2"]["b"][None, :],
            se["l1"]["W"], se["l1"]["b"][None, :], se["l2"]["W"], se["l2"]["b"][None, :],
            ue["l1"]["W"], ue["l1"]["b"][None, :], ue["l2"]["W"], ue["l2"]["b"][None, :],
            wd, bde, wnc, bnc]
    return pl.pallas_call(
        _prep_u_body,
        out_shape=outs,
        in_specs=[_full(a.shape) for a in args],
        out_specs=(_full((B, H)), _full((3, B, H)), _full((3, B, H))),
    )(*args)


def _node_enc_body(x_ref, oneh_ref, w1, b1, w2, b2, wa, wb, te,
                   h_ref, a_ref, bv_ref):
    h = jnp.maximum(x_ref[...] @ w1[...] + b1[...], 0.0) @ w2[...] + b2[...]
    h_ref[...] = h
    a_ref[...] = h @ wa[...] + oneh_ref[...] @ te[...]
    bv_ref[...] = h @ wb[...]


def _tc_node_enc(x, oneh, p, wa1, wb1, te1):
    ne = p["node_enc"]
    args = [x, oneh, ne["l1"]["W"], ne["l1"]["b"][None, :],
            ne["l2"]["W"], ne["l2"]["b"][None, :], wa1, wb1, te1]
    grid = (N // BN,)
    in_specs = [pl.BlockSpec((BN, x.shape[1]), lambda i: (i, 0)),
                pl.BlockSpec((BN, B), lambda i: (i, 0))] + \
               [_full(a.shape) for a in args[2:]]
    outs = tuple(jax.ShapeDtypeStruct((N, H), jnp.float32) for _ in range(3))
    out_specs = tuple(pl.BlockSpec((BN, H), lambda i: (i, 0)) for _ in range(3))
    return pl.pallas_call(_node_enc_body, grid=grid, out_shape=outs,
                          in_specs=in_specs, out_specs=out_specs)(*args)


def _edge0_body(g_ref, ea_ref, ew1, eb1, ew2, eb2, wc, w2, b2, out_ref):
    e0 = jnp.maximum(ea_ref[...] @ ew1[...] + eb1[...], 0.0) @ ew2[...] + eb2[...]
    z = jnp.maximum(g_ref[...] + e0 @ wc[...], 0.0)
    out_ref[...] = z @ w2[...] + b2[...]


def _tc_edge0(g, edge_attr, p, wc, w2, b2):
    ee = p["edge_enc"]
    args = [g, edge_attr, ee["l1"]["W"], ee["l1"]["b"][None, :],
            ee["l2"]["W"], ee["l2"]["b"][None, :], wc, w2, b2[None, :]]
    grid = (E // BE,)
    in_specs = [pl.BlockSpec((BE, H), lambda i: (i, 0)),
                pl.BlockSpec((BE, edge_attr.shape[1]), lambda i: (i, 0))] + \
               [_full(a.shape) for a in args[2:]]
    return pl.pallas_call(
        _edge0_body, grid=grid,
        out_shape=jax.ShapeDtypeStruct((E, H), jnp.float32),
        in_specs=in_specs,
        out_specs=pl.BlockSpec((BE, H), lambda i: (i, 0)))(*args)


def _edgeL_body(g_ref, e_ref, wc, w2, b2, out_ref):
    z = jnp.maximum(g_ref[...] + e_ref[...] @ wc[...], 0.0)
    out_ref[...] = z @ w2[...] + b2[...]


def _tc_edgeL(g, e, wc, w2, b2):
    args = [g, e, wc, w2, b2[None, :]]
    grid = (E // BE,)
    in_specs = [pl.BlockSpec((BE, H), lambda i: (i, 0)),
                pl.BlockSpec((BE, H), lambda i: (i, 0))] + \
               [_full(a.shape) for a in args[2:]]
    return pl.pallas_call(
        _edgeL_body, grid=grid,
        out_shape=jax.ShapeDtypeStruct((E, H), jnp.float32),
        in_specs=in_specs,
        out_specs=pl.BlockSpec((BE, H), lambda i: (i, 0)))(*args)


def _node_stage_body(h_ref, ns_ref, deg_ref, oneh_ref, wna, wnb, tn, wn2, bn2,
                     wa, wb, te, h_out, a_out, bv_out):
    deg = deg_ref[:, 0:1] + deg_ref[:, 16:17]
    inv = 1.0 / jnp.maximum(deg, 1.0)
    agg = ns_ref[...] * inv
    h = h_ref[...]
    zn = jnp.maximum(h @ wna[...] + agg @ wnb[...] + oneh_ref[...] @ tn[...], 0.0)
    hn = zn @ wn2[...] + bn2[...] + h
    h_out[...] = hn
    a_out[...] = hn @ wa[...] + oneh_ref[...] @ te[...]
    bv_out[...] = hn @ wb[...]


def _node_last_body(h_ref, ns_ref, deg_ref, oneh_ref, wna, wnb, tn, wn2, bn2,
                    h_out):
    deg = deg_ref[:, 0:1] + deg_ref[:, 16:17]
    inv = 1.0 / jnp.maximum(deg, 1.0)
    agg = ns_ref[...] * inv
    h = h_ref[...]
    zn = jnp.maximum(h @ wna[...] + agg @ wnb[...] + oneh_ref[...] @ tn[...], 0.0)
    h_out[...] = zn @ wn2[...] + bn2[...] + h


def _tc_node_stage(h, node_sum, degs, oneh, lw, next_w):
    wna, wnb, tn, wn2, bn2 = lw
    base_args = [h, node_sum, degs, oneh, wna, wnb, tn, wn2, bn2[None, :]]
    grid = (N // BN,)
    base_specs = [pl.BlockSpec((BN, H), lambda i: (i, 0)),
                  pl.BlockSpec((BN, H), lambda i: (i, 0)),
                  pl.BlockSpec((BN, 32), lambda i: (i, 0)),
                  pl.BlockSpec((BN, B), lambda i: (i, 0))] + \
                 [_full(a.shape) for a in base_args[4:]]
    if next_w is None:
        return pl.pallas_call(
            _node_last_body, grid=grid,
            out_shape=jax.ShapeDtypeStruct((N, H), jnp.float32),
            in_specs=base_specs,
            out_specs=pl.BlockSpec((BN, H), lambda i: (i, 0)))(*base_args)
    wa, wb, te = next_w
    args = base_args + [wa, wb, te]
    in_specs = base_specs + [_full(a.shape) for a in (wa, wb, te)]
    outs = tuple(jax.ShapeDtypeStruct((N, H), jnp.float32) for _ in range(3))
    out_specs = tuple(pl.BlockSpec((BN, H), lambda i: (i, 0)) for _ in range(3))
    return pl.pallas_call(_node_stage_body, grid=grid, out_shape=outs,
                          in_specs=in_specs, out_specs=out_specs)(*args)


def _pool_body(h_ref, ns_ref, deg_ref, oneh_ref, d1, db1, d2, db2,
               xout_ref, acc_ref):
    h = h_ref[...]
    xout_ref[...] = jnp.maximum(h @ d1[...] + db1[...], 0.0) @ d2[...] + db2[...]
    oneh = oneh_ref[...]
    deg = deg_ref[:, 0:1] + deg_ref[:, 16:17]
    hsum = _dotT(oneh, h)
    esum = _dotT(oneh, ns_ref[...])
    ncnt = _dotT(oneh, jnp.ones_like(h))
    ecnt = _dotT(oneh, jnp.broadcast_to(deg, h.shape))
    acc = jnp.stack([hsum, esum, ncnt, ecnt], axis=0)

    @pl.when(pl.program_id(0) == 0)
    def _():
        acc_ref[...] = acc

    @pl.when(pl.program_id(0) != 0)
    def _():
        acc_ref[...] = acc_ref[...] + acc


def _tc_pool(h, node_sum, degs, oneh, p):
    nd = p["node_dec"]
    args = [h, node_sum, degs, oneh, nd["l1"]["W"], nd["l1"]["b"][None, :],
            nd["l2"]["W"], nd["l2"]["b"][None, :]]
    grid = (N // BN,)
    in_specs = [pl.BlockSpec((BN, H), lambda i: (i, 0)),
                pl.BlockSpec((BN, H), lambda i: (i, 0)),
                pl.BlockSpec((BN, 32), lambda i: (i, 0)),
                pl.BlockSpec((BN, B), lambda i: (i, 0))] + \
               [_full(a.shape) for a in args[4:]]
    nout = nd["l2"]["W"].shape[1]
    outs = (jax.ShapeDtypeStruct((N, nout), jnp.float32),
            jax.ShapeDtypeStruct((4, B, H), jnp.float32))
    out_specs = (pl.BlockSpec((BN, nout), lambda i: (i, 0)),
                 pl.BlockSpec((4, B, H), lambda i: (0, 0, 0)))
    return pl.pallas_call(_pool_body, grid=grid, out_shape=outs,
                          in_specs=in_specs, out_specs=out_specs)(*args)


def _head_body(acc_ref, u_ref, hw1, hb1, hw2, hb2, out_ref):
    ng = acc_ref[0] / jnp.maximum(acc_ref[2], 1.0)
    eg = acc_ref[1] / jnp.maximum(acc_ref[3], 1.0)
    cat = jnp.concatenate([ng, eg, u_ref[...]], axis=1)
    out_ref[...] = (jnp.maximum(cat @ hw1[...] + hb1[...], 0.0)
                    @ hw2[...] + hb2[...])


def _tc_head(acc, u, p):
    hd = p["head"]
    args = [acc, u, hd["l1"]["W"], hd["l1"]["b"][None, :],
            hd["l2"]["W"], hd["l2"]["b"][None, :]]
    ld = hd["l2"]["W"].shape[1]
    return pl.pallas_call(
        _head_body,
        out_shape=jax.ShapeDtypeStruct((B, ld), jnp.float32),
        in_specs=[_full(a.shape) for a in args],
        out_specs=_full((B, ld)))(*args)


def kernel(x, edge_index, edge_attr, conditions, scale, batch, params):
    p = params
    row = edge_index[0].astype(jnp.int32)
    col = edge_index[1].astype(jnp.int32)
    oneh = (batch[:, None] == jnp.arange(B, dtype=batch.dtype)[None, :]
            ).astype(jnp.float32)

    was, wbs, wcs, wds, bde = [], [], [], [], []
    wnas, wnbs, wncs, bnc = [], [], [], []
    for lp in p["layers"]:
        w1 = lp["edge"]["l1"]["W"]
        was.append(w1[:H]); wbs.append(w1[H:2 * H])
        wcs.append(w1[2 * H:3 * H]); wds.append(w1[3 * H:])
        bde.append(lp["edge"]["l1"]["b"])
        wn1 = lp["node"]["l1"]["W"]
        wnas.append(wn1[:H]); wnbs.append(wn1[H:2 * H]); wncs.append(wn1[2 * H:])
        bnc.append(lp["node"]["l1"]["b"])
    wd = jnp.stack(wds); bde_s = jnp.stack(bde)
    wnc = jnp.stack(wncs); bnc_s = jnp.stack(bnc)

    u, te, tn = _tc_prep_u(conditions, scale, p, wd, bde_s, wnc, bnc_s)

    degs = _sc_degree(row)

    h, a, bv = _tc_node_enc(x, oneh, p, was[0], wbs[0], te[0])

    e = None
    node_sum = None
    for li in range(3):
        lp = p["layers"][li]
        g = _sc_gather(a, bv, row, col)
        if li == 0:
            e = _tc_edge0(g, edge_attr, p, wcs[0], lp["edge"]["l2"]["W"],
                          lp["edge"]["l2"]["b"])
        else:
            e = _tc_edgeL(g, e, wcs[li], lp["edge"]["l2"]["W"],
                          lp["edge"]["l2"]["b"])
        node_sum = _sc_scatter(e, row)
        lw = (wnas[li], wnbs[li], tn[li], lp["node"]["l2"]["W"],
              lp["node"]["l2"]["b"])
        if li < 2:
            h, a, bv = _tc_node_stage(h, node_sum, degs, oneh, lw,
                                      (was[li + 1], wbs[li + 1], te[li + 1]))
        else:
            h = _tc_node_stage(h, node_sum, degs, oneh, lw, None)

    x_out, acc = _tc_pool(h, node_sum, degs, oneh, p)
    log_ratios = _tc_head(acc, u, p)
    return (x_out, log_ratios)


# speedup vs baseline: 3.6429x; 1.2217x over previous
import jax
import jax.numpy as jnp
from jax import lax
from jax.experimental import pallas as pl
from jax.experimental.pallas import tpu as pltpu
from jax.experimental.pallas import tpu_sc as plsc

N = 50000
E = 800000
B = 16
H = 64

NC = 2
NS = 16
NW = NC * NS

CHUNK = 128
NCHUNKS = E // CHUNK
RPT = 3128
NPAD = NS * RPT

BN = 5000
BE = 4000


def _sc_mesh():
    return plsc.VectorSubcoreMesh(core_axis_name="c", subcore_axis_name="s")


_SC_PARAMS = pltpu.CompilerParams(use_tc_tiling_on_sc=False)


def _gather_body(nchunks, a_hbm, b_hbm, row_hbm, col_hbm, out_hbm,
                 idxr_v, idxc_v, ar_v, bc_v, sem1, sem2):
    wid = lax.axis_index("s") * NC + lax.axis_index("c")
    nk = (nchunks + NW - 1) // NW

    def step(k, _):
        j = wid + k * NW

        @pl.when(j < nchunks)
        def _():
            off = j * CHUNK
            pltpu.sync_copy(row_hbm.at[pl.ds(off, CHUNK)], idxr_v)
            pltpu.sync_copy(col_hbm.at[pl.ds(off, CHUNK)], idxc_v)
            cp1 = pltpu.async_copy(a_hbm.at[idxr_v], ar_v, sem1)
            cp2 = pltpu.async_copy(b_hbm.at[idxc_v], bc_v, sem2)
            cp1.wait()
            cp2.wait()

            def addrow(i, _):
                for q in range(H // 16):
                    sl = (i, pl.ds(q * 16, 16))
                    ar_v[sl] = ar_v[sl] + bc_v[sl]
                return 0

            lax.fori_loop(0, CHUNK, addrow, 0)
            pltpu.sync_copy(ar_v, out_hbm.at[pl.ds(off, CHUNK)])

        return 0

    lax.fori_loop(0, nk, step, 0)


def _sc_gather(a, b, row, col):
    ne = row.shape[0]
    import functools as _ft
    k = pl.kernel(
        _ft.partial(_gather_body, ne // CHUNK),
        mesh=_sc_mesh(),
        compiler_params=_SC_PARAMS,
        out_type=jax.ShapeDtypeStruct((ne, H), jnp.float32),
        scratch_types=[
            pltpu.VMEM((CHUNK,), jnp.int32),
            pltpu.VMEM((CHUNK,), jnp.int32),
            pltpu.VMEM((CHUNK, H), jnp.float32),
            pltpu.VMEM((CHUNK, H), jnp.float32),
            pltpu.SemaphoreType.DMA,
            pltpu.SemaphoreType.DMA,
        ],
    )
    return k(a, b, row, col)


def _scatter_body(nchunks, e_hbm, row_hbm, out_hbm, acc_sh, stage_v, idx_v,
                  vals_v):
    c = lax.axis_index("c")
    s = lax.axis_index("s")
    nk = (nchunks + NS - 1) // NS

    for q in range(2):
        qi = c * 2 + q

        def zrow(i, _):
            stage_v[(i, pl.ds(0, 16))] = jnp.zeros((16,), jnp.float32)
            return 0

        lax.fori_loop(0, RPT, zrow, 0)
        pltpu.sync_copy(stage_v, acc_sh.at[pl.ds(s * RPT, RPT)])
        plsc.subcore_barrier()

        def step(k, _):
            j = s + k * NS

            @pl.when(j < nchunks)
            def _():
                off = j * CHUNK
                pltpu.sync_copy(row_hbm.at[pl.ds(off, CHUNK)], idx_v)
                pltpu.sync_copy(
                    e_hbm.at[pl.ds(off, CHUNK), pl.ds(qi * 16, 16)], vals_v)
                pltpu.sync_copy(vals_v, acc_sh.at[idx_v], add=True)

            return 0

        lax.fori_loop(0, nk, step, 0)
        plsc.subcore_barrier()
        pltpu.sync_copy(acc_sh.at[pl.ds(s * RPT, RPT)], stage_v)
        pltpu.sync_copy(stage_v,
                        out_hbm.at[pl.ds(s * RPT, RPT), pl.ds(qi * 16, 16)])
        plsc.subcore_barrier()


def _sc_scatter(e, row):
    import functools as _ft
    k = pl.kernel(
        _ft.partial(_scatter_body, row.shape[0] // CHUNK),
        mesh=_sc_mesh(),
        compiler_params=_SC_PARAMS,
        out_type=jax.ShapeDtypeStruct((NPAD, H), jnp.float32),
        scratch_types=[
            pltpu.VMEM_SHARED((NPAD, 16), jnp.float32),
            pltpu.VMEM((RPT, 16), jnp.float32),
            pltpu.VMEM((CHUNK,), jnp.int32),
            pltpu.VMEM((CHUNK, 16), jnp.float32),
        ],
    )
    return k(e, row)[:N]


def _deg_body(row_hbm, out_hbm, acc_sh, stage_v, idx_v, ones_v):
    c = lax.axis_index("c")
    s = lax.axis_index("s")
    wid = s * NC + c

    def zrow(i, _):
        stage_v[(i, pl.ds(0, 16))] = jnp.zeros((16,), jnp.float32)
        return 0

    lax.fori_loop(0, RPT, zrow, 0)
    pltpu.sync_copy(stage_v, acc_sh.at[pl.ds(s * RPT, RPT)])

    def orow(i, _):
        ones_v[(i, pl.ds(0, 16))] = jnp.ones((16,), jnp.float32)
        return 0

    lax.fori_loop(0, CHUNK, orow, 0)
    plsc.subcore_barrier()

    nk = (NCHUNKS + NW - 1) // NW

    def step(k, _):
        j = wid + k * NW

        @pl.when(j < NCHUNKS)
        def _():
            off = j * CHUNK
            pltpu.sync_copy(row_hbm.at[pl.ds(off, CHUNK)], idx_v)
            pltpu.sync_copy(ones_v, acc_sh.at[idx_v], add=True)

        return 0

    lax.fori_loop(0, nk, step, 0)
    plsc.subcore_barrier()
    pltpu.sync_copy(acc_sh.at[pl.ds(s * RPT, RPT)], stage_v)
    pltpu.sync_copy(stage_v,
                    out_hbm.at[pl.ds(s * RPT, RPT), pl.ds(c * 16, 16)])


def _sc_degree(row):
    k = pl.kernel(
        _deg_body,
        mesh=_sc_mesh(),
        compiler_params=_SC_PARAMS,
        out_type=jax.ShapeDtypeStruct((NPAD, 32), jnp.float32),
        scratch_types=[
            pltpu.VMEM_SHARED((NPAD, 16), jnp.float32),
            pltpu.VMEM((RPT, 16), jnp.float32),
            pltpu.VMEM((CHUNK,), jnp.int32),
            pltpu.VMEM((CHUNK, 16), jnp.float32),
        ],
    )
    return k(row)[:N]


def _full(shape):
    return pl.BlockSpec(shape, lambda *_: tuple(0 for _ in shape))


def _dotT(a, b):
    return lax.dot_general(a, b, (((0,), (0,)), ((), ())),
                           preferred_element_type=jnp.float32)


def _prep_u_body(cond_ref, scl_ref, cw1, cb1, cw2, cb2, sw1, sb1, sw2, sb2,
                 uw1, ub1, uw2, ub2, wd_ref, bde_ref, wnc_ref, bnc_ref,
                 u_ref, te_ref, tn_ref):
    uc = jnp.maximum(cond_ref[...] @ cw1[...] + cb1[...], 0.0) @ cw2[...] + cb2[...]
    us = jnp.maximum(scl_ref[...] @ sw1[...] + sb1[...], 0.0) @ sw2[...] + sb2[...]
    cat = jnp.concatenate([uc, us], axis=1)
    u = jnp.maximum(cat @ uw1[...] + ub1[...], 0.0) @ uw2[...] + ub2[...]
    u_ref[...] = u
    for l in range(3):
        te_ref[l] = u @ wd_ref[l] + bde_ref[l]
        tn_ref[l] = u @ wnc_ref[l] + bnc_ref[l]


def _tc_prep_u(conditions, scale, p, wd, bde, wnc, bnc):
    outs = (
        jax.ShapeDtypeStruct((B, H), jnp.float32),
        jax.ShapeDtypeStruct((3, B, H), jnp.float32),
        jax.ShapeDtypeStruct((3, B, H), jnp.float32),
    )
    ce, se, ue = p["cond_enc"], p["scale_enc"], p["u_enc"]
    args = [conditions, scale,
            ce["l1"]["W"], ce["l1"]["b"][None, :], ce["l2"]["W"], ce["l2"]["b"][None, :],
            se["l1"]["W"], se["l1"]["b"][None, :], se["l2"]["W"], se["l2"]["b"][None, :],
            ue["l1"]["W"], ue["l1"]["b"][None, :], ue["l2"]["W"], ue["l2"]["b"][None, :],
            wd, bde, wnc, bnc]
    return pl.pallas_call(
        _prep_u_body,
        out_shape=outs,
        in_specs=[_full(a.shape) for a in args],
        out_specs=(_full((B, H)), _full((3, B, H)), _full((3, B, H))),
    )(*args)


def _node_enc_body(x_ref, oneh_ref, w1, b1, w2, b2, wa, wb, te,
                   h_ref, a_ref, bv_ref):
    h = jnp.maximum(x_ref[...] @ w1[...] + b1[...], 0.0) @ w2[...] + b2[...]
    h_ref[...] = h
    a_ref[...] = h @ wa[...] + oneh_ref[...] @ te[...]
    bv_ref[...] = h @ wb[...]


def _tc_node_enc(x, oneh, p, wa1, wb1, te1):
    ne = p["node_enc"]
    args = [x, oneh, ne["l1"]["W"], ne["l1"]["b"][None, :],
            ne["l2"]["W"], ne["l2"]["b"][None, :], wa1, wb1, te1]
    grid = (N // BN,)
    in_specs = [pl.BlockSpec((BN, x.shape[1]), lambda i: (i, 0)),
                pl.BlockSpec((BN, B), lambda i: (i, 0))] + \
               [_full(a.shape) for a in args[2:]]
    outs = tuple(jax.ShapeDtypeStruct((N, H), jnp.float32) for _ in range(3))
    out_specs = tuple(pl.BlockSpec((BN, H), lambda i: (i, 0)) for _ in range(3))
    return pl.pallas_call(_node_enc_body, grid=grid, out_shape=outs,
                          in_specs=in_specs, out_specs=out_specs)(*args)


def _edge0_body(g_ref, ea_ref, ew1, eb1, ew2, eb2, wc, w2, b2, out_ref):
    e0 = jnp.maximum(ea_ref[...] @ ew1[...] + eb1[...], 0.0) @ ew2[...] + eb2[...]
    z = jnp.maximum(g_ref[...] + e0 @ wc[...], 0.0)
    out_ref[...] = z @ w2[...] + b2[...]


def _tc_edge0(g, edge_attr, p, wc, w2, b2):
    ee = p["edge_enc"]
    args = [g, edge_attr, ee["l1"]["W"], ee["l1"]["b"][None, :],
            ee["l2"]["W"], ee["l2"]["b"][None, :], wc, w2, b2[None, :]]
    ne = g.shape[0]
    grid = (ne // BE,)
    in_specs = [pl.BlockSpec((BE, H), lambda i: (i, 0)),
                pl.BlockSpec((BE, edge_attr.shape[1]), lambda i: (i, 0))] + \
               [_full(a.shape) for a in args[2:]]
    return pl.pallas_call(
        _edge0_body, grid=grid,
        out_shape=jax.ShapeDtypeStruct((ne, H), jnp.float32),
        in_specs=in_specs,
        out_specs=pl.BlockSpec((BE, H), lambda i: (i, 0)))(*args)


def _edgeL_body(g_ref, e_ref, wc, w2, b2, out_ref):
    z = jnp.maximum(g_ref[...] + e_ref[...] @ wc[...], 0.0)
    out_ref[...] = z @ w2[...] + b2[...]


def _tc_edgeL(g, e, wc, w2, b2):
    args = [g, e, wc, w2, b2[None, :]]
    ne = g.shape[0]
    grid = (ne // BE,)
    in_specs = [pl.BlockSpec((BE, H), lambda i: (i, 0)),
                pl.BlockSpec((BE, H), lambda i: (i, 0))] + \
               [_full(a.shape) for a in args[2:]]
    return pl.pallas_call(
        _edgeL_body, grid=grid,
        out_shape=jax.ShapeDtypeStruct((ne, H), jnp.float32),
        in_specs=in_specs,
        out_specs=pl.BlockSpec((BE, H), lambda i: (i, 0)))(*args)


def _node_stage_body(h_ref, ns1_ref, ns2_ref, deg_ref, oneh_ref, wna, wnb, tn,
                     wn2, bn2, wa, wb, te, h_out, a_out, bv_out):
    deg = deg_ref[:, 0:1] + deg_ref[:, 16:17]
    inv = 1.0 / jnp.maximum(deg, 1.0)
    agg = (ns1_ref[...] + ns2_ref[...]) * inv
    h = h_ref[...]
    zn = jnp.maximum(h @ wna[...] + agg @ wnb[...] + oneh_ref[...] @ tn[...], 0.0)
    hn = zn @ wn2[...] + bn2[...] + h
    h_out[...] = hn
    a_out[...] = hn @ wa[...] + oneh_ref[...] @ te[...]
    bv_out[...] = hn @ wb[...]


def _node_last_body(h_ref, ns1_ref, ns2_ref, deg_ref, oneh_ref, wna, wnb, tn,
                    wn2, bn2, h_out):
    deg = deg_ref[:, 0:1] + deg_ref[:, 16:17]
    inv = 1.0 / jnp.maximum(deg, 1.0)
    agg = (ns1_ref[...] + ns2_ref[...]) * inv
    h = h_ref[...]
    zn = jnp.maximum(h @ wna[...] + agg @ wnb[...] + oneh_ref[...] @ tn[...], 0.0)
    h_out[...] = zn @ wn2[...] + bn2[...] + h


def _tc_node_stage(h, ns1, ns2, degs, oneh, lw, next_w):
    wna, wnb, tn, wn2, bn2 = lw
    base_args = [h, ns1, ns2, degs, oneh, wna, wnb, tn, wn2, bn2[None, :]]
    grid = (N // BN,)
    base_specs = [pl.BlockSpec((BN, H), lambda i: (i, 0)),
                  pl.BlockSpec((BN, H), lambda i: (i, 0)),
                  pl.BlockSpec((BN, H), lambda i: (i, 0)),
                  pl.BlockSpec((BN, 32), lambda i: (i, 0)),
                  pl.BlockSpec((BN, B), lambda i: (i, 0))] + \
                 [_full(a.shape) for a in base_args[5:]]
    if next_w is None:
        return pl.pallas_call(
            _node_last_body, grid=grid,
            out_shape=jax.ShapeDtypeStruct((N, H), jnp.float32),
            in_specs=base_specs,
            out_specs=pl.BlockSpec((BN, H), lambda i: (i, 0)))(*base_args)
    wa, wb, te = next_w
    args = base_args + [wa, wb, te]
    in_specs = base_specs + [_full(a.shape) for a in (wa, wb, te)]
    outs = tuple(jax.ShapeDtypeStruct((N, H), jnp.float32) for _ in range(3))
    out_specs = tuple(pl.BlockSpec((BN, H), lambda i: (i, 0)) for _ in range(3))
    return pl.pallas_call(_node_stage_body, grid=grid, out_shape=outs,
                          in_specs=in_specs, out_specs=out_specs)(*args)


def _pool_body(h_ref, ns1_ref, ns2_ref, deg_ref, oneh_ref, d1, db1, d2, db2,
               xout_ref, acc_ref):
    h = h_ref[...]
    xout_ref[...] = jnp.maximum(h @ d1[...] + db1[...], 0.0) @ d2[...] + db2[...]
    oneh = oneh_ref[...]
    deg = deg_ref[:, 0:1] + deg_ref[:, 16:17]
    hsum = _dotT(oneh, h)
    esum = _dotT(oneh, ns1_ref[...] + ns2_ref[...])
    ncnt = _dotT(oneh, jnp.ones_like(h))
    ecnt = _dotT(oneh, jnp.broadcast_to(deg, h.shape))
    acc = jnp.stack([hsum, esum, ncnt, ecnt], axis=0)

    @pl.when(pl.program_id(0) == 0)
    def _():
        acc_ref[...] = acc

    @pl.when(pl.program_id(0) != 0)
    def _():
        acc_ref[...] = acc_ref[...] + acc


def _tc_pool(h, ns1, ns2, degs, oneh, p):
    nd = p["node_dec"]
    args = [h, ns1, ns2, degs, oneh, nd["l1"]["W"], nd["l1"]["b"][None, :],
            nd["l2"]["W"], nd["l2"]["b"][None, :]]
    grid = (N // BN,)
    in_specs = [pl.BlockSpec((BN, H), lambda i: (i, 0)),
                pl.BlockSpec((BN, H), lambda i: (i, 0)),
                pl.BlockSpec((BN, H), lambda i: (i, 0)),
                pl.BlockSpec((BN, 32), lambda i: (i, 0)),
                pl.BlockSpec((BN, B), lambda i: (i, 0))] + \
               [_full(a.shape) for a in args[5:]]
    nout = nd["l2"]["W"].shape[1]
    outs = (jax.ShapeDtypeStruct((N, nout), jnp.float32),
            jax.ShapeDtypeStruct((4, B, H), jnp.float32))
    out_specs = (pl.BlockSpec((BN, nout), lambda i: (i, 0)),
                 pl.BlockSpec((4, B, H), lambda i: (0, 0, 0)))
    return pl.pallas_call(_pool_body, grid=grid, out_shape=outs,
                          in_specs=in_specs, out_specs=out_specs)(*args)


def _head_body(acc_ref, u_ref, hw1, hb1, hw2, hb2, out_ref):
    ng = acc_ref[0] / jnp.maximum(acc_ref[2], 1.0)
    eg = acc_ref[1] / jnp.maximum(acc_ref[3], 1.0)
    cat = jnp.concatenate([ng, eg, u_ref[...]], axis=1)
    out_ref[...] = (jnp.maximum(cat @ hw1[...] + hb1[...], 0.0)
                    @ hw2[...] + hb2[...])


def _tc_head(acc, u, p):
    hd = p["head"]
    args = [acc, u, hd["l1"]["W"], hd["l1"]["b"][None, :],
            hd["l2"]["W"], hd["l2"]["b"][None, :]]
    ld = hd["l2"]["W"].shape[1]
    return pl.pallas_call(
        _head_body,
        out_shape=jax.ShapeDtypeStruct((B, ld), jnp.float32),
        in_specs=[_full(a.shape) for a in args],
        out_specs=_full((B, ld)))(*args)


def kernel(x, edge_index, edge_attr, conditions, scale, batch, params):
    p = params
    row = edge_index[0].astype(jnp.int32)
    col = edge_index[1].astype(jnp.int32)
    oneh = (batch[:, None] == jnp.arange(B, dtype=batch.dtype)[None, :]
            ).astype(jnp.float32)

    was, wbs, wcs, wds, bde = [], [], [], [], []
    wnas, wnbs, wncs, bnc = [], [], [], []
    for lp in p["layers"]:
        w1 = lp["edge"]["l1"]["W"]
        was.append(w1[:H]); wbs.append(w1[H:2 * H])
        wcs.append(w1[2 * H:3 * H]); wds.append(w1[3 * H:])
        bde.append(lp["edge"]["l1"]["b"])
        wn1 = lp["node"]["l1"]["W"]
        wnas.append(wn1[:H]); wnbs.append(wn1[H:2 * H]); wncs.append(wn1[2 * H:])
        bnc.append(lp["node"]["l1"]["b"])
    wd = jnp.stack(wds); bde_s = jnp.stack(bde)
    wnc = jnp.stack(wncs); bnc_s = jnp.stack(bnc)

    u, te, tn = _tc_prep_u(conditions, scale, p, wd, bde_s, wnc, bnc_s)

    degs = _sc_degree(row)

    h, a, bv = _tc_node_enc(x, oneh, p, was[0], wbs[0], te[0])

    E2 = E // 2
    rows = (row[:E2], row[E2:])
    cols = (col[:E2], col[E2:])
    eattrs = (edge_attr[:E2], edge_attr[E2:])

    es = [None, None]
    nss = [None, None]
    for li in range(3):
        lp = p["layers"][li]
        w2e, b2e = lp["edge"]["l2"]["W"], lp["edge"]["l2"]["b"]
        gs = [_sc_gather(a, bv, rows[hh], cols[hh]) for hh in range(2)]
        for hh in range(2):
            if li == 0:
                es[hh] = _tc_edge0(gs[hh], eattrs[hh], p, wcs[0], w2e, b2e)
            else:
                es[hh] = _tc_edgeL(gs[hh], es[hh], wcs[li], w2e, b2e)
            nss[hh] = _sc_scatter(es[hh], rows[hh])
        lw = (wnas[li], wnbs[li], tn[li], lp["node"]["l2"]["W"],
              lp["node"]["l2"]["b"])
        if li < 2:
            h, a, bv = _tc_node_stage(h, nss[0], nss[1], degs, oneh, lw,
                                      (was[li + 1], wbs[li + 1], te[li + 1]))
        else:
            h = _tc_node_stage(h, nss[0], nss[1], degs, oneh, lw, None)

    x_out, acc = _tc_pool(h, nss[0], nss[1], degs, oneh, p)
    log_ratios = _tc_head(acc, u, p)
    return (x_out, log_ratios)


# speedup vs baseline: 4.7142x; 1.2941x over previous
import jax
import jax.numpy as jnp
from jax import lax
from jax.experimental import pallas as pl
from jax.experimental.pallas import tpu as pltpu
from jax.experimental.pallas import tpu_sc as plsc

N = 50000
E = 800000
B = 16
H = 64

NC = 2
NS = 16
NW = NC * NS

CHUNK = 128
NCHUNKS = E // CHUNK
RPT = 3128
NPAD = NS * RPT

BN = 5000
BE = 4096


def _sc_mesh():
    return plsc.VectorSubcoreMesh(core_axis_name="c", subcore_axis_name="s")


_SC_PARAMS = pltpu.CompilerParams(use_tc_tiling_on_sc=False)


def _gather_body(nkw, a_hbm, b_hbm, row_hbm, col_hbm, out_hbm,
                 idxr_v, idxc_v, ar_v, bc_v, sem_i, sem_g, sem_w):
    wid = lax.axis_index("s") * NC + lax.axis_index("c")

    def off(t):
        return (wid + t * NW) * CHUNK

    def idx_load(t, b):
        pltpu.async_copy(row_hbm.at[pl.ds(off(t), CHUNK)], idxr_v.at[b], sem_i)
        pltpu.async_copy(col_hbm.at[pl.ds(off(t), CHUNK)], idxc_v.at[b], sem_i)

    def idx_wait(b):
        pltpu.make_async_copy(row_hbm.at[pl.ds(0, CHUNK)], idxr_v.at[b],
                              sem_i).wait()
        pltpu.make_async_copy(col_hbm.at[pl.ds(0, CHUNK)], idxc_v.at[b],
                              sem_i).wait()

    def gathers_start(b):
        pltpu.async_copy(a_hbm.at[idxr_v.at[b]], ar_v.at[b], sem_g)
        pltpu.async_copy(b_hbm.at[idxc_v.at[b]], bc_v.at[b], sem_g)

    def gathers_wait(b):
        pltpu.make_async_copy(a_hbm.at[idxr_v.at[b]], ar_v.at[b], sem_g).wait()
        pltpu.make_async_copy(b_hbm.at[idxc_v.at[b]], bc_v.at[b], sem_g).wait()

    def wb_start(t, b):
        pltpu.async_copy(ar_v.at[b], out_hbm.at[pl.ds(off(t), CHUNK)], sem_w)

    def wb_wait(t, b):
        pltpu.make_async_copy(ar_v.at[b], out_hbm.at[pl.ds(off(t), CHUNK)],
                              sem_w).wait()

    idx_load(0, 0)
    idx_wait(0)
    gathers_start(0)
    idx_load(1, 1)

    def super_step(k2, _):
        for b in (0, 1):
            t = k2 * 2 + b

            @pl.when(t >= 1)
            def _():
                wb_wait(t - 1, 1 - b)

            @pl.when(t + 1 < nkw)
            def _():
                idx_wait(1 - b)
                gathers_start(1 - b)

            gathers_wait(b)

            @pl.when(t + 2 < nkw)
            def _():
                idx_load(t + 2, b)

            def addrow(i, _):
                for q in range(H // 16):
                    sl = (b, i, pl.ds(q * 16, 16))
                    ar_v[sl] = ar_v[sl] + bc_v[sl]
                return 0

            lax.fori_loop(0, CHUNK, addrow, 0)
            wb_start(t, b)
        return 0

    lax.fori_loop(0, nkw // 2, super_step, 0)
    wb_wait(nkw - 1, 1)


def _sc_gather(a, b, row, col):
    ne = row.shape[0]
    import functools as _ft
    k = pl.kernel(
        _ft.partial(_gather_body, ne // CHUNK // NW),
        mesh=_sc_mesh(),
        compiler_params=_SC_PARAMS,
        out_type=jax.ShapeDtypeStruct((ne, H), jnp.float32),
        scratch_types=[
            pltpu.VMEM((2, CHUNK), jnp.int32),
            pltpu.VMEM((2, CHUNK), jnp.int32),
            pltpu.VMEM((2, CHUNK, H), jnp.float32),
            pltpu.VMEM((2, CHUNK, H), jnp.float32),
            pltpu.SemaphoreType.DMA,
            pltpu.SemaphoreType.DMA,
            pltpu.SemaphoreType.DMA,
        ],
    )
    return k(a, b, row, col)


def _scatter_body(nkt, e_hbm, row_hbm, out_hbm, acc_sh, stage_v, idx_v,
                  vals_v, sem_i, sem_v, sem_a):
    c = lax.axis_index("c")
    s = lax.axis_index("s")

    def off(t):
        return (s + t * NS) * CHUNK

    for q in range(2):
        qi = c * 2 + q

        def zrow(i, _):
            stage_v[(i, pl.ds(0, 16))] = jnp.zeros((16,), jnp.float32)
            return 0

        lax.fori_loop(0, RPT, zrow, 0)
        pltpu.sync_copy(stage_v, acc_sh.at[pl.ds(s * RPT, RPT)])
        plsc.subcore_barrier()

        def loads(t, b):
            pltpu.async_copy(row_hbm.at[pl.ds(off(t), CHUNK)], idx_v.at[b],
                             sem_i)
            pltpu.async_copy(
                e_hbm.at[pl.ds(off(t), CHUNK), pl.ds(qi * 16, 16)],
                vals_v.at[b], sem_v)

        def loads_wait(b):
            pltpu.make_async_copy(row_hbm.at[pl.ds(0, CHUNK)], idx_v.at[b],
                                  sem_i).wait()
            pltpu.make_async_copy(
                e_hbm.at[pl.ds(0, CHUNK), pl.ds(qi * 16, 16)],
                vals_v.at[b], sem_v).wait()

        def add_start(b):
            pltpu.async_copy(vals_v.at[b], acc_sh.at[idx_v.at[b]], sem_a,
                             add=True)

        def add_wait(b):
            pltpu.make_async_copy(vals_v.at[b], acc_sh.at[idx_v.at[b]],
                                  sem_a).wait()

        loads(0, 0)

        def super_step(k2, _):
            for b in (0, 1):
                t = k2 * 2 + b

                @pl.when(t >= 1)
                def _():
                    add_wait(1 - b)

                @pl.when(t + 1 < nkt)
                def _():
                    loads(t + 1, 1 - b)

                loads_wait(b)
                add_start(b)
            return 0

        lax.fori_loop(0, nkt // 2, super_step, 0)
        add_wait(1)
        plsc.subcore_barrier()
        pltpu.sync_copy(acc_sh.at[pl.ds(s * RPT, RPT)], stage_v)
        pltpu.sync_copy(stage_v,
                        out_hbm.at[pl.ds(s * RPT, RPT), pl.ds(qi * 16, 16)])
        plsc.subcore_barrier()


def _sc_scatter(e, row):
    import functools as _ft
    k = pl.kernel(
        _ft.partial(_scatter_body, row.shape[0] // CHUNK // NS),
        mesh=_sc_mesh(),
        compiler_params=_SC_PARAMS,
        out_type=jax.ShapeDtypeStruct((NPAD, H), jnp.float32),
        scratch_types=[
            pltpu.VMEM_SHARED((NPAD, 16), jnp.float32),
            pltpu.VMEM((RPT, 16), jnp.float32),
            pltpu.VMEM((2, CHUNK), jnp.int32),
            pltpu.VMEM((2, CHUNK, 16), jnp.float32),
            pltpu.SemaphoreType.DMA,
            pltpu.SemaphoreType.DMA,
            pltpu.SemaphoreType.DMA,
        ],
    )
    return k(e, row)[:N]


def _deg_body(row_hbm, out_hbm, acc_sh, stage_v, idx_v, ones_v):
    c = lax.axis_index("c")
    s = lax.axis_index("s")
    wid = s * NC + c

    def zrow(i, _):
        stage_v[(i, pl.ds(0, 16))] = jnp.zeros((16,), jnp.float32)
        return 0

    lax.fori_loop(0, RPT, zrow, 0)
    pltpu.sync_copy(stage_v, acc_sh.at[pl.ds(s * RPT, RPT)])

    def orow(i, _):
        ones_v[(i, pl.ds(0, 16))] = jnp.ones((16,), jnp.float32)
        return 0

    lax.fori_loop(0, CHUNK, orow, 0)
    plsc.subcore_barrier()

    nk = (NCHUNKS + NW - 1) // NW

    def step(k, _):
        j = wid + k * NW

        @pl.when(j < NCHUNKS)
        def _():
            off = j * CHUNK
            pltpu.sync_copy(row_hbm.at[pl.ds(off, CHUNK)], idx_v)
            pltpu.sync_copy(ones_v, acc_sh.at[idx_v], add=True)

        return 0

    lax.fori_loop(0, nk, step, 0)
    plsc.subcore_barrier()
    pltpu.sync_copy(acc_sh.at[pl.ds(s * RPT, RPT)], stage_v)
    pltpu.sync_copy(stage_v,
                    out_hbm.at[pl.ds(s * RPT, RPT), pl.ds(c * 16, 16)])


def _sc_degree(row):
    k = pl.kernel(
        _deg_body,
        mesh=_sc_mesh(),
        compiler_params=_SC_PARAMS,
        out_type=jax.ShapeDtypeStruct((NPAD, 32), jnp.float32),
        scratch_types=[
            pltpu.VMEM_SHARED((NPAD, 16), jnp.float32),
            pltpu.VMEM((RPT, 16), jnp.float32),
            pltpu.VMEM((CHUNK,), jnp.int32),
            pltpu.VMEM((CHUNK, 16), jnp.float32),
        ],
    )
    return k(row)[:N]


def _full(shape):
    return pl.BlockSpec(shape, lambda *_: tuple(0 for _ in shape))


def _dotT(a, b):
    return lax.dot_general(a, b, (((0,), (0,)), ((), ())),
                           preferred_element_type=jnp.float32)


def _prep_u_body(cond_ref, scl_ref, cw1, cb1, cw2, cb2, sw1, sb1, sw2, sb2,
                 uw1, ub1, uw2, ub2, wd_ref, bde_ref, wnc_ref, bnc_ref,
                 u_ref, te_ref, tn_ref):
    uc = jnp.maximum(cond_ref[...] @ cw1[...] + cb1[...], 0.0) @ cw2[...] + cb2[...]
    us = jnp.maximum(scl_ref[...] @ sw1[...] + sb1[...], 0.0) @ sw2[...] + sb2[...]
    cat = jnp.concatenate([uc, us], axis=1)
    u = jnp.maximum(cat @ uw1[...] + ub1[...], 0.0) @ uw2[...] + ub2[...]
    u_ref[...] = u
    for l in range(3):
        te_ref[l] = u @ wd_ref[l] + bde_ref[l]
        tn_ref[l] = u @ wnc_ref[l] + bnc_ref[l]


def _tc_prep_u(conditions, scale, p, wd, bde, wnc, bnc):
    outs = (
        jax.ShapeDtypeStruct((B, H), jnp.float32),
        jax.ShapeDtypeStruct((3, B, H), jnp.float32),
        jax.ShapeDtypeStruct((3, B, H), jnp.float32),
    )
    ce, se, ue = p["cond_enc"], p["scale_enc"], p["u_enc"]
    args = [conditions, scale,
            ce["l1"]["W"], ce["l1"]["b"][None, :], ce["l2"]["W"], ce["l2"]["b"][None, :],
            se["l1"]["W"], se["l1"]["b"][None, :], se["l2"]["W"], se["l2"]["b"][None, :],
            ue["l1"]["W"], ue["l1"]["b"][None, :], ue["l2"]["W"], ue["l2"]["b"][None, :],
            wd, bde, wnc, bnc]
    return pl.pallas_call(
        _prep_u_body,
        out_shape=outs,
        in_specs=[_full(a.shape) for a in args],
        out_specs=(_full((B, H)), _full((3, B, H)), _full((3, B, H))),
    )(*args)


def _node_enc_body(x_ref, oneh_ref, w1, b1, w2, b2, wa, wb, te,
                   h_ref, a_ref, bv_ref):
    h = jnp.maximum(x_ref[...] @ w1[...] + b1[...], 0.0) @ w2[...] + b2[...]
    h_ref[...] = h
    a_ref[...] = h @ wa[...] + oneh_ref[...] @ te[...]
    bv_ref[...] = h @ wb[...]


def _tc_node_enc(x, oneh, p, wa1, wb1, te1):
    ne = p["node_enc"]
    args = [x, oneh, ne["l1"]["W"], ne["l1"]["b"][None, :],
            ne["l2"]["W"], ne["l2"]["b"][None, :], wa1, wb1, te1]
    grid = (N // BN,)
    in_specs = [pl.BlockSpec((BN, x.shape[1]), lambda i: (i, 0)),
                pl.BlockSpec((BN, B), lambda i: (i, 0))] + \
               [_full(a.shape) for a in args[2:]]
    outs = tuple(jax.ShapeDtypeStruct((N, H), jnp.float32) for _ in range(3))
    out_specs = tuple(pl.BlockSpec((BN, H), lambda i: (i, 0)) for _ in range(3))
    return pl.pallas_call(_node_enc_body, grid=grid, out_shape=outs,
                          in_specs=in_specs, out_specs=out_specs)(*args)


def _edge0_body(g_ref, ea_ref, ew1, eb1, ew2, eb2, wc, w2, b2, out_ref):
    e0 = jnp.maximum(ea_ref[...] @ ew1[...] + eb1[...], 0.0) @ ew2[...] + eb2[...]
    z = jnp.maximum(g_ref[...] + e0 @ wc[...], 0.0)
    out_ref[...] = z @ w2[...] + b2[...]


def _tc_edge0(g, edge_attr, p, wc, w2, b2):
    ee = p["edge_enc"]
    args = [g, edge_attr, ee["l1"]["W"], ee["l1"]["b"][None, :],
            ee["l2"]["W"], ee["l2"]["b"][None, :], wc, w2, b2[None, :]]
    ne = g.shape[0]
    grid = (ne // BE,)
    in_specs = [pl.BlockSpec((BE, H), lambda i: (i, 0)),
                pl.BlockSpec((BE, edge_attr.shape[1]), lambda i: (i, 0))] + \
               [_full(a.shape) for a in args[2:]]
    return pl.pallas_call(
        _edge0_body, grid=grid,
        out_shape=jax.ShapeDtypeStruct((ne, H), jnp.float32),
        in_specs=in_specs,
        out_specs=pl.BlockSpec((BE, H), lambda i: (i, 0)))(*args)


def _edgeL_body(g_ref, e_ref, wc, w2, b2, out_ref):
    z = jnp.maximum(g_ref[...] + e_ref[...] @ wc[...], 0.0)
    out_ref[...] = z @ w2[...] + b2[...]


def _tc_edgeL(g, e, wc, w2, b2):
    args = [g, e, wc, w2, b2[None, :]]
    ne = g.shape[0]
    grid = (ne // BE,)
    in_specs = [pl.BlockSpec((BE, H), lambda i: (i, 0)),
                pl.BlockSpec((BE, H), lambda i: (i, 0))] + \
               [_full(a.shape) for a in args[2:]]
    return pl.pallas_call(
        _edgeL_body, grid=grid,
        out_shape=jax.ShapeDtypeStruct((ne, H), jnp.float32),
        in_specs=in_specs,
        out_specs=pl.BlockSpec((BE, H), lambda i: (i, 0)))(*args)


def _node_stage_body(h_ref, ns1_ref, ns2_ref, deg_ref, oneh_ref, wna, wnb, tn,
                     wn2, bn2, wa, wb, te, h_out, a_out, bv_out):
    deg = deg_ref[:, 0:1] + deg_ref[:, 16:17]
    inv = 1.0 / jnp.maximum(deg, 1.0)
    agg = (ns1_ref[...] + ns2_ref[...]) * inv
    h = h_ref[...]
    zn = jnp.maximum(h @ wna[...] + agg @ wnb[...] + oneh_ref[...] @ tn[...], 0.0)
    hn = zn @ wn2[...] + bn2[...] + h
    h_out[...] = hn
    a_out[...] = hn @ wa[...] + oneh_ref[...] @ te[...]
    bv_out[...] = hn @ wb[...]


def _node_last_body(h_ref, ns1_ref, ns2_ref, deg_ref, oneh_ref, wna, wnb, tn,
                    wn2, bn2, h_out):
    deg = deg_ref[:, 0:1] + deg_ref[:, 16:17]
    inv = 1.0 / jnp.maximum(deg, 1.0)
    agg = (ns1_ref[...] + ns2_ref[...]) * inv
    h = h_ref[...]
    zn = jnp.maximum(h @ wna[...] + agg @ wnb[...] + oneh_ref[...] @ tn[...], 0.0)
    h_out[...] = zn @ wn2[...] + bn2[...] + h


def _tc_node_stage(h, ns1, ns2, degs, oneh, lw, next_w):
    wna, wnb, tn, wn2, bn2 = lw
    base_args = [h, ns1, ns2, degs, oneh, wna, wnb, tn, wn2, bn2[None, :]]
    grid = (N // BN,)
    base_specs = [pl.BlockSpec((BN, H), lambda i: (i, 0)),
                  pl.BlockSpec((BN, H), lambda i: (i, 0)),
                  pl.BlockSpec((BN, H), lambda i: (i, 0)),
                  pl.BlockSpec((BN, 32), lambda i: (i, 0)),
                  pl.BlockSpec((BN, B), lambda i: (i, 0))] + \
                 [_full(a.shape) for a in base_args[5:]]
    if next_w is None:
        return pl.pallas_call(
            _node_last_body, grid=grid,
            out_shape=jax.ShapeDtypeStruct((N, H), jnp.float32),
            in_specs=base_specs,
            out_specs=pl.BlockSpec((BN, H), lambda i: (i, 0)))(*base_args)
    wa, wb, te = next_w
    args = base_args + [wa, wb, te]
    in_specs = base_specs + [_full(a.shape) for a in (wa, wb, te)]
    outs = tuple(jax.ShapeDtypeStruct((N, H), jnp.float32) for _ in range(3))
    out_specs = tuple(pl.BlockSpec((BN, H), lambda i: (i, 0)) for _ in range(3))
    return pl.pallas_call(_node_stage_body, grid=grid, out_shape=outs,
                          in_specs=in_specs, out_specs=out_specs)(*args)


def _pool_body(h_ref, ns1_ref, ns2_ref, deg_ref, oneh_ref, d1, db1, d2, db2,
               xout_ref, acc_ref):
    h = h_ref[...]
    xout_ref[...] = jnp.maximum(h @ d1[...] + db1[...], 0.0) @ d2[...] + db2[...]
    oneh = oneh_ref[...]
    deg = deg_ref[:, 0:1] + deg_ref[:, 16:17]
    hsum = _dotT(oneh, h)
    esum = _dotT(oneh, ns1_ref[...] + ns2_ref[...])
    ncnt = _dotT(oneh, jnp.ones_like(h))
    ecnt = _dotT(oneh, jnp.broadcast_to(deg, h.shape))
    acc = jnp.stack([hsum, esum, ncnt, ecnt], axis=0)

    @pl.when(pl.program_id(0) == 0)
    def _():
        acc_ref[...] = acc

    @pl.when(pl.program_id(0) != 0)
    def _():
        acc_ref[...] = acc_ref[...] + acc


def _tc_pool(h, ns1, ns2, degs, oneh, p):
    nd = p["node_dec"]
    args = [h, ns1, ns2, degs, oneh, nd["l1"]["W"], nd["l1"]["b"][None, :],
            nd["l2"]["W"], nd["l2"]["b"][None, :]]
    grid = (N // BN,)
    in_specs = [pl.BlockSpec((BN, H), lambda i: (i, 0)),
                pl.BlockSpec((BN, H), lambda i: (i, 0)),
                pl.BlockSpec((BN, H), lambda i: (i, 0)),
                pl.BlockSpec((BN, 32), lambda i: (i, 0)),
                pl.BlockSpec((BN, B), lambda i: (i, 0))] + \
               [_full(a.shape) for a in args[5:]]
    nout = nd["l2"]["W"].shape[1]
    outs = (jax.ShapeDtypeStruct((N, nout), jnp.float32),
            jax.ShapeDtypeStruct((4, B, H), jnp.float32))
    out_specs = (pl.BlockSpec((BN, nout), lambda i: (i, 0)),
                 pl.BlockSpec((4, B, H), lambda i: (0, 0, 0)))
    return pl.pallas_call(_pool_body, grid=grid, out_shape=outs,
                          in_specs=in_specs, out_specs=out_specs)(*args)


def _head_body(acc_ref, u_ref, hw1, hb1, hw2, hb2, out_ref):
    ng = acc_ref[0] / jnp.maximum(acc_ref[2], 1.0)
    eg = acc_ref[1] / jnp.maximum(acc_ref[3], 1.0)
    cat = jnp.concatenate([ng, eg, u_ref[...]], axis=1)
    out_ref[...] = (jnp.maximum(cat @ hw1[...] + hb1[...], 0.0)
                    @ hw2[...] + hb2[...])


def _tc_head(acc, u, p):
    hd = p["head"]
    args = [acc, u, hd["l1"]["W"], hd["l1"]["b"][None, :],
            hd["l2"]["W"], hd["l2"]["b"][None, :]]
    ld = hd["l2"]["W"].shape[1]
    return pl.pallas_call(
        _head_body,
        out_shape=jax.ShapeDtypeStruct((B, ld), jnp.float32),
        in_specs=[_full(a.shape) for a in args],
        out_specs=_full((B, ld)))(*args)


def kernel(x, edge_index, edge_attr, conditions, scale, batch, params):
    p = params
    row = edge_index[0].astype(jnp.int32)
    col = edge_index[1].astype(jnp.int32)
    oneh = (batch[:, None] == jnp.arange(B, dtype=batch.dtype)[None, :]
            ).astype(jnp.float32)

    was, wbs, wcs, wds, bde = [], [], [], [], []
    wnas, wnbs, wncs, bnc = [], [], [], []
    for lp in p["layers"]:
        w1 = lp["edge"]["l1"]["W"]
        was.append(w1[:H]); wbs.append(w1[H:2 * H])
        wcs.append(w1[2 * H:3 * H]); wds.append(w1[3 * H:])
        bde.append(lp["edge"]["l1"]["b"])
        wn1 = lp["node"]["l1"]["W"]
        wnas.append(wn1[:H]); wnbs.append(wn1[H:2 * H]); wncs.append(wn1[2 * H:])
        bnc.append(lp["node"]["l1"]["b"])
    wd = jnp.stack(wds); bde_s = jnp.stack(bde)
    wnc = jnp.stack(wncs); bnc_s = jnp.stack(bnc)

    u, te, tn = _tc_prep_u(conditions, scale, p, wd, bde_s, wnc, bnc_s)

    degs = _sc_degree(row)

    h, a, bv = _tc_node_enc(x, oneh, p, was[0], wbs[0], te[0])

    E2 = E // 2
    E2P = ((E2 + NW * CHUNK - 1) // (NW * CHUNK)) * (NW * CHUNK)
    padn = E2P - E2
    zpad = jnp.zeros((padn,), jnp.int32)
    dpad = jnp.full((padn,), NPAD - 1, jnp.int32)
    apad = jnp.zeros((padn, edge_attr.shape[1]), jnp.float32)
    rows = tuple(jnp.concatenate([r, zpad]) for r in (row[:E2], row[E2:]))
    rows_s = tuple(jnp.concatenate([r, dpad]) for r in (row[:E2], row[E2:]))
    cols = tuple(jnp.concatenate([r, zpad]) for r in (col[:E2], col[E2:]))
    eattrs = tuple(jnp.concatenate([ea, apad])
                   for ea in (edge_attr[:E2], edge_attr[E2:]))

    es = [None, None]
    nss = [None, None]
    for li in range(3):
        lp = p["layers"][li]
        w2e, b2e = lp["edge"]["l2"]["W"], lp["edge"]["l2"]["b"]
        gs = [_sc_gather(a, bv, rows[hh], cols[hh]) for hh in range(2)]
        for hh in range(2):
            if li == 0:
                es[hh] = _tc_edge0(gs[hh], eattrs[hh], p, wcs[0], w2e, b2e)
            else:
                es[hh] = _tc_edgeL(gs[hh], es[hh], wcs[li], w2e, b2e)
            nss[hh] = _sc_scatter(es[hh], rows_s[hh])
        lw = (wnas[li], wnbs[li], tn[li], lp["node"]["l2"]["W"],
              lp["node"]["l2"]["b"])
        if li < 2:
            h, a, bv = _tc_node_stage(h, nss[0], nss[1], degs, oneh, lw,
                                      (was[li + 1], wbs[li + 1], te[li + 1]))
        else:
            h = _tc_node_stage(h, nss[0], nss[1], degs, oneh, lw, None)

    x_out, acc = _tc_pool(h, nss[0], nss[1], degs, oneh, p)
    log_ratios = _tc_head(acc, u, p)
    return (x_out, log_ratios)
